# Initial kernel scaffold; baseline (speedup 1.0000x reference)
import jax
import jax.numpy as jnp
from jax.experimental import pallas as pl


def kernel(xyz):
    raise NotImplementedError("write your pallas kernel here")


# speedup vs baseline: 28.1613x; 28.1613x over previous
import functools

import jax
import jax.numpy as jnp
from jax import lax
from jax.experimental import pallas as pl
from jax.experimental.pallas import tpu as pltpu

B = 16
N = 8192
M = 1024


def _fps_kernel(x_ref, y_ref, z_ref, cx_ref, cy_ref, cz_ref, d_ref):
    d_ref[...] = jnp.full((B, N), jnp.inf, dtype=jnp.float32)
    iota = lax.broadcasted_iota(jnp.int32, (B, N), 1)

    def body(k, carry):
        fx, fy, fz = carry
        cx_ref[pl.ds(k, 1), :] = fx.reshape(1, B)
        cy_ref[pl.ds(k, 1), :] = fy.reshape(1, B)
        cz_ref[pl.ds(k, 1), :] = fz.reshape(1, B)

        dx = x_ref[...] - fx
        dy = y_ref[...] - fy
        dz = z_ref[...] - fz
        dist = (dx * dx + dz * dz) + dy * dy
        d = jnp.minimum(d_ref[...], dist)
        d_ref[...] = d

        m = jnp.max(d, axis=1, keepdims=True)
        cand = jnp.where(d == m, iota, N)
        j = jnp.min(cand, axis=1, keepdims=True)
        onehot = iota == j
        zero = jnp.zeros((B, N), dtype=jnp.float32)
        nfx = jnp.sum(jnp.where(onehot, x_ref[...], zero), axis=1, keepdims=True)
        nfy = jnp.sum(jnp.where(onehot, y_ref[...], zero), axis=1, keepdims=True)
        nfz = jnp.sum(jnp.where(onehot, z_ref[...], zero), axis=1, keepdims=True)
        return nfx, nfy, nfz

    init = (x_ref[:, 0:1], y_ref[:, 0:1], z_ref[:, 0:1])
    lax.fori_loop(0, M, body, init)


@jax.jit
def kernel(xyz):
    x = xyz[:, :, 0]
    y = xyz[:, :, 1]
    z = xyz[:, :, 2]
    out_shape = jax.ShapeDtypeStruct((M, B), jnp.float32)
    cx, cy, cz = pl.pallas_call(
        _fps_kernel,
        out_shape=(out_shape, out_shape, out_shape),
        scratch_shapes=[pltpu.VMEM((B, N), jnp.float32)],
    )(x, y, z)
    return jnp.stack([cx.T, cy.T, cz.T], axis=-1)
